# baseline (speedup 1.0000x reference)
import functools
import numpy as np
import jax
import jax.numpy as jnp
from jax import lax
from jax.experimental import pallas as pl
from jax.experimental.pallas import tpu as pltpu
from jax.experimental.pallas import tpu_sc as plsc

B, L, K = 16, 640, 16
NS, NV = 128, 16
ES, EV = 32, 1
NLAYERS = 3
N = B * L
E = N * K
NB = 128
EB = NB * K
GRID = N // NB
PACK = NS + 3 * NV
TW = 256
EFW = 48
LB = 128
CHUNK = 128
NWORK = 32


def _sc_gather(table, idx2d):
    nchunks, _ = idx2d.shape
    D = table.shape[1]
    cpw = nchunks // NWORK
    mesh = plsc.VectorSubcoreMesh(core_axis_name="c", subcore_axis_name="s")

    @functools.partial(
        pl.kernel, mesh=mesh,
        out_type=jax.ShapeDtypeStruct((nchunks * CHUNK, D), jnp.float32),
        scratch_types=[
            pltpu.VMEM((CHUNK,), jnp.int32),
            pltpu.VMEM((CHUNK, D), jnp.float32),
            pltpu.SemaphoreType.DMA,
        ],
    )
    def k(table_hbm, idx_hbm, out_hbm, idx_v, rows_v, sem):
        wid = lax.axis_index("s") * 2 + lax.axis_index("c")

        def body(i, carry):
            chunk = wid * cpw + i
            pltpu.sync_copy(idx_hbm.at[chunk], idx_v)
            pltpu.async_copy(table_hbm.at[idx_v], rows_v, sem).wait()
            pltpu.sync_copy(rows_v, out_hbm.at[pl.ds(chunk * CHUNK, CHUNK)])
            return carry

        lax.fori_loop(0, cpw, body, 0)

    return k(table, idx2d)


def _full(x):
    nd = x.ndim
    return pl.BlockSpec(x.shape, lambda i, _nd=nd: (0,) * _nd)


def _rep(x):
    d = x.shape[-1]
    return jnp.broadcast_to(x[:, None, :], (NB, K, d)).reshape(NB * K, d)


def _sumk(x):
    d = x.shape[-1]
    return jnp.sum(x.reshape(NB, K, d), axis=1)


def _ln(x, g, b):
    mu = jnp.mean(x, axis=1, keepdims=True)
    xc = x - mu
    var = jnp.mean(xc * xc, axis=1, keepdims=True)
    return g * xc / jnp.sqrt(var + 1e-4) + b


def _lnv(v):
    s2 = v[0] * v[0] + v[1] * v[1] + v[2] * v[2]
    vn = jnp.sqrt(jnp.mean(s2, axis=1, keepdims=True) + 1e-4)
    return [vc / vn for vc in v]


def _knn_body(ca_ref, cat_ref, mr_ref, mc_ref, o_ref):
    ca = ca_ref[0]
    cat = cat_ref[0]
    r = pl.program_id(1)
    d2 = jnp.zeros((LB, L), jnp.float32)
    for c in range(3):
        d = ca[:, c:c + 1] - cat[c:c + 1, :]
        d2 = d2 + d * d
    valid = mr_ref[0] * mc_ref[0]
    rowi = lax.broadcasted_iota(jnp.int32, (LB, L), 0) + r * LB
    coli = lax.broadcasted_iota(jnp.int32, (LB, L), 1)
    cur = jnp.where(valid > 0, d2, 1e10) + jnp.where(coli == rowi, 1e10, 0.0)
    cols = []
    for _ in range(K):
        m = jnp.min(cur, axis=1, keepdims=True)
        idx = jnp.min(jnp.where(cur <= m, coli, L), axis=1, keepdims=True)
        cols.append(idx)
        cur = jnp.where(coli == idx, jnp.float32(3e10), cur)
    o_ref[...] = jnp.concatenate(cols, axis=1)


def _knn(ca4, caT, mrow, mcol):
    rb = L // LB
    return pl.pallas_call(
        _knn_body,
        grid=(B, rb),
        in_specs=[
            pl.BlockSpec((1, LB, 4), lambda b, r: (b, r, 0)),
            pl.BlockSpec((1, 4, L), lambda b, r: (b, 0, 0)),
            pl.BlockSpec((1, LB, 1), lambda b, r: (b, r, 0)),
            pl.BlockSpec((1, 1, L), lambda b, r: (b, 0, 0)),
        ],
        out_specs=pl.BlockSpec((LB, K), lambda b, r, _rb=rb: (b * _rb + r, 0)),
        out_shape=jax.ShapeDtypeStruct((N, K), jnp.int32),
    )(ca4, caT, mrow, mcol)


def _edge_body(cad_ref, g0_ref, ang_ref, wh_ref, wv_ref, wss_ref, wsv_ref,
               bs_ref, wg_ref, bg_ref, o_ref):
    cad = _rep(cad_ref[...])
    cas = g0_ref[...][:, PACK:PACK + 3]
    dc = [cas[:, c:c + 1] - cad[:, c:c + 1] for c in range(3)]
    dist = jnp.sqrt(dc[0] * dc[0] + dc[1] * dc[1] + dc[2] * dc[2])
    ev = [d / (dist + 1e-8) for d in dc]
    mu = lax.broadcasted_iota(jnp.int32, (1, 16), 1).astype(jnp.float32) * (
        20.0 / 15.0)
    rbf = jnp.exp(-(((dist - mu) / 1.25) ** 2))
    tvec = lax.broadcasted_iota(jnp.int32, (1, 8), 1).astype(jnp.float32) * 2.0
    freq = jnp.exp(tvec * (-np.log(10000.0) / 16.0))
    ang = ang_ref[...] * freq
    pe = jnp.concatenate([jnp.cos(ang), jnp.sin(ang)], 1)
    es0 = jnp.concatenate([rbf, pe], 1)
    vh = [e * wh_ref[...] for e in ev]
    vn = jnp.sqrt(vh[0] * vh[0] + vh[1] * vh[1] + vh[2] * vh[2] + 1e-8)
    so = jnp.dot(es0, wss_ref[...]) + vn * wsv_ref[...] + bs_ref[...]
    gate = jax.nn.sigmoid(
        jnp.sum(so * wg_ref[...], axis=1, keepdims=True) + bg_ref[...])
    vo = [v * wv_ref[...] * gate for v in vh]
    o_ref[...] = jnp.concatenate(
        [so, vo[0], vo[1], vo[2], jnp.zeros((EB, EFW - 35), jnp.float32)], 1)


def _edge_features(ca16, g0, ang8, ep):
    Ws = ep['Ws']
    wts = [ep['Wh'], ep['Wv'], Ws[:ES], Ws[ES:ES + 1], ep['bs'][None],
           ep['Wg'].T, ep['bg'][None]]
    return pl.pallas_call(
        _edge_body,
        grid=(GRID,),
        in_specs=[
            pl.BlockSpec((NB, 16), lambda i: (i, 0)),
            pl.BlockSpec((EB, TW), lambda i: (i, 0)),
            pl.BlockSpec((EB, 8), lambda i: (i, 0)),
        ] + [_full(w) for w in wts],
        out_specs=pl.BlockSpec((EB, EFW), lambda i: (i, 0)),
        out_shape=jax.ShapeDtypeStruct((E, EFW), jnp.float32),
    )(ca16, g0, ang8, *wts)


def _node_body(ns_ref, nv_ref, ca_ref, wh_ref, wv_ref, wss_ref, wsv_ref,
               bs_ref, wg_ref, bg_ref, g0_ref, b0_ref, o_ref):
    ns = ns_ref[...][:, :7]
    v = [nv_ref[...][:, 3 * c:3 * c + 3] for c in range(3)]
    vh = [jnp.dot(vc, wh_ref[...]) for vc in v]
    vn = jnp.sqrt(vh[0] * vh[0] + vh[1] * vh[1] + vh[2] * vh[2] + 1e-8)
    so = jnp.dot(ns, wss_ref[...]) + jnp.dot(vn, wsv_ref[...]) + bs_ref[...]
    vu = [jnp.dot(vhc, wv_ref[...]) for vhc in vh]
    gate = jax.nn.sigmoid(jnp.dot(so, wg_ref[...]) + bg_ref[...])
    vo = [u * gate for u in vu]
    hs = _ln(so, g0_ref[...], b0_ref[...])
    hv = _lnv(vo)
    ca = ca_ref[...][:, :3]
    pad = jnp.zeros((NB, TW - PACK - 3), jnp.float32)
    o_ref[...] = jnp.concatenate([hs] + hv + [ca, pad], 1)


def _node_embed(ns8, nv16, ca16, np_, ln0):
    Ws = np_['Ws']
    wts = [np_['Wh'], np_['Wv'], Ws[:7], Ws[7:], np_['bs'][None],
           np_['Wg'], np_['bg'][None], ln0['g'][None], ln0['b'][None]]
    return pl.pallas_call(
        _node_body,
        grid=(GRID,),
        in_specs=[
            pl.BlockSpec((NB, 8), lambda i: (i, 0)),
            pl.BlockSpec((NB, 16), lambda i: (i, 0)),
            pl.BlockSpec((NB, 16), lambda i: (i, 0)),
        ] + [_full(w) for w in wts],
        out_specs=pl.BlockSpec((NB, TW), lambda i: (i, 0)),
        out_shape=jax.ShapeDtypeStruct((N, TW), jnp.float32),
    )(ns8, nv16, ca16, *wts)


def _prep_layer(lp):
    w = {}
    m0 = lp['msg'][0]
    Wh, Ws = m0['Wh'], m0['Ws']
    w['m0_Wh_d'] = Wh[:NV]
    w['m0_Wh_s'] = Wh[NV:2 * NV]
    w['m0_wh_e'] = Wh[2 * NV:]
    w['m0_Ws_d'] = Ws[:NS]
    w['m0_Ws_s'] = Ws[NS:2 * NS]
    w['m0_Ws_e'] = Ws[2 * NS:2 * NS + ES]
    w['m0_Ws_v'] = Ws[2 * NS + ES:]
    w['m0_bs'] = m0['bs'][None]
    w['m0_Wv'] = m0['Wv']
    w['m0_Wg'] = m0['Wg']
    w['m0_bg'] = m0['bg'][None]
    for i in (1, 2):
        m, p = lp['msg'][i], f'm{i}'
        w[p + '_Wh'] = m['Wh']
        w[p + '_Ws_s'] = m['Ws'][:NS]
        w[p + '_Ws_v'] = m['Ws'][NS:]
        w[p + '_bs'] = m['bs'][None]
        w[p + '_Wv'] = m['Wv']
        w[p + '_Wg'] = m['Wg']
        w[p + '_bg'] = m['bg'][None]
    for i, (p, si) in enumerate((('f0', NS), ('f1', 2 * NS))):
        m = lp['ff'][i]
        w[p + '_Wh'] = m['Wh']
        w[p + '_Ws_s'] = m['Ws'][:si]
        w[p + '_Ws_v'] = m['Ws'][si:]
        w[p + '_bs'] = m['bs'][None]
        w[p + '_Wv'] = m['Wv']
        w[p + '_Wg'] = m['Wg']
        w[p + '_bg'] = m['bg'][None]
    w['ln1_g'] = lp['ln1']['g'][None]
    w['ln1_b'] = lp['ln1']['b'][None]
    w['ln2_g'] = lp['ln2']['g'][None]
    w['ln2_b'] = lp['ln2']['b'][None]
    for nm in w:
        if ('_Wh' in nm and nm != 'm0_wh_e') or '_Ws_' in nm \
                or nm.endswith('_Wv') or nm.endswith('_Wg'):
            w[nm] = w[nm].astype(jnp.bfloat16)
    return w


def _make_layer_body(wnames, last):
    def body(*refs):
        o_ref = refs[-1]
        tdst = refs[0][...]
        tsrc = refs[1][...]
        ef = refs[2][...]
        W = {nm: refs[3 + i][...] for i, nm in enumerate(wnames)}
        r_ref = refs[3 + len(wnames)] if last else None

        def bdot(a, b):
            return jnp.dot(a.astype(jnp.bfloat16), b,
                           preferred_element_type=jnp.float32)

        def gvp(s, v, p, act):
            vh = [bdot(vc, W[p + '_Wh']) for vc in v]
            vn = jnp.sqrt(vh[0] * vh[0] + vh[1] * vh[1] + vh[2] * vh[2] + 1e-8)
            so = (bdot(s, W[p + '_Ws_s']) + bdot(vn, W[p + '_Ws_v'])
                  + W[p + '_bs'])
            vu = [bdot(vhc, W[p + '_Wv']) for vhc in vh]
            gate = jax.nn.sigmoid(bdot(so, W[p + '_Wg']) + W[p + '_bg'])
            vo = [u * gate for u in vu]
            if act:
                so = jax.nn.relu(so)
            return so, vo

        hs_d = tdst[:, :NS]
        hv_d = [tdst[:, NS + NV * c:NS + NV * (c + 1)] for c in range(3)]
        hs_s = tsrc[:, :NS]
        hv_s = [tsrc[:, NS + NV * c:NS + NV * (c + 1)] for c in range(3)]
        es = ef[:, :ES]
        ev = [ef[:, ES + c:ES + c + 1] for c in range(3)]

        vh = [_rep(bdot(hv_d[c], W['m0_Wh_d'])) + bdot(hv_s[c], W['m0_Wh_s'])
              + ev[c] * W['m0_wh_e'] for c in range(3)]
        vn = jnp.sqrt(vh[0] * vh[0] + vh[1] * vh[1] + vh[2] * vh[2] + 1e-8)
        so = (_rep(bdot(hs_d, W['m0_Ws_d'])) + bdot(hs_s, W['m0_Ws_s'])
              + bdot(es, W['m0_Ws_e']) + bdot(vn, W['m0_Ws_v'])
              + W['m0_bs'])
        vu = [bdot(vhc, W['m0_Wv']) for vhc in vh]
        gate = jax.nn.sigmoid(bdot(so, W['m0_Wg']) + W['m0_bg'])
        v = [u * gate for u in vu]
        s = jax.nn.relu(so)

        s, v = gvp(s, v, 'm1', True)
        s, v = gvp(s, v, 'm2', False)

        h1 = _ln(hs_d + _sumk(s) * (1.0 / K), W['ln1_g'], W['ln1_b'])
        w1 = _lnv([hv_d[c] + _sumk(v[c]) * (1.0 / K) for c in range(3)])

        fs, fv = gvp(h1, w1, 'f0', True)
        fs, fv = gvp(fs, fv, 'f1', False)
        h2 = _ln(h1 + fs, W['ln2_g'], W['ln2_b'])
        w2 = _lnv([w1[c] + fv[c] for c in range(3)])

        pad = jnp.zeros((NB, TW - PACK), jnp.float32)
        if last:
            R = r_ref[...]
            vr = [w2[0] * R[:, 0 + j:1 + j] + w2[1] * R[:, 3 + j:4 + j]
                  + w2[2] * R[:, 6 + j:7 + j] for j in range(3)]
            o_ref[...] = jnp.concatenate([h2] + vr + [pad], 1)
        else:
            o_ref[...] = jnp.concatenate([h2] + w2 + [pad], 1)

    return body


def _layer(tdst, tsrc, ef, lp, r16, last):
    w = _prep_layer(lp)
    wnames = list(w.keys())
    wvals = [w[nm] for nm in wnames]
    ins = [tdst, tsrc, ef] + wvals
    specs = [
        pl.BlockSpec((NB, TW), lambda i: (i, 0)),
        pl.BlockSpec((EB, TW), lambda i: (i, 0)),
        pl.BlockSpec((EB, EFW), lambda i: (i, 0)),
    ] + [_full(x) for x in wvals]
    if last:
        ins.append(r16)
        specs.append(pl.BlockSpec((NB, 16), lambda i: (i, 0)))
    return pl.pallas_call(
        _make_layer_body(wnames, last),
        grid=(GRID,),
        in_specs=specs,
        out_specs=pl.BlockSpec((NB, TW), lambda i: (i, 0)),
        out_shape=jax.ShapeDtypeStruct((N, TW), jnp.float32),
    )(*ins)


def _norm(v, eps=1e-8):
    return v / (jnp.linalg.norm(v, axis=-1, keepdims=True) + eps)


def _dihedrals(coords, eps=1e-7):
    X = coords.reshape(coords.shape[0], -1, 3)
    dX = X[:, 1:] - X[:, :-1]
    U = _norm(dX)
    u2, u1, u0 = U[:, :-2], U[:, 1:-1], U[:, 2:]
    n2 = _norm(jnp.cross(u2, u1))
    n1 = _norm(jnp.cross(u1, u0))
    cosD = jnp.clip(jnp.sum(n2 * n1, -1), -1 + eps, 1 - eps)
    D = jnp.sign(jnp.sum(u2 * n1, -1)) * jnp.arccos(cosD)
    D = jnp.pad(D, ((0, 0), (1, 2)))
    D = D.reshape(coords.shape[0], -1, 3)
    return jnp.concatenate([jnp.cos(D), jnp.sin(D)], -1)


def _rot_frames(coords):
    n, ca, c = coords[:, :, 0], coords[:, :, 1], coords[:, :, 2]
    e1 = _norm(c - ca)
    v2 = n - ca
    u2 = v2 - e1 * jnp.sum(e1 * v2, -1, keepdims=True)
    e2 = _norm(u2)
    e3 = jnp.cross(e1, e2)
    return jnp.stack([e1, e2, e3], axis=-2)


def kernel(coords, coord_mask, padding_mask, confidence, params):
    mask = (coord_mask & (~padding_mask)).astype(jnp.float32)
    CA = coords[:, :, 1, :]

    dih = _dihedrals(coords)
    node_s = jnp.concatenate([dih, confidence[..., None]], -1) * mask[..., None]
    fwdv = jnp.pad(_norm(CA[:, 1:] - CA[:, :-1]), ((0, 0), (0, 1), (0, 0)))
    bwdv = jnp.pad(_norm(CA[:, :-1] - CA[:, 1:]), ((0, 0), (1, 0), (0, 0)))
    c_v = _norm(coords[:, :, 2] - CA)
    n_v = _norm(coords[:, :, 0] - CA)
    bis = _norm(c_v + n_v)
    perp = _norm(jnp.cross(c_v, n_v))
    side = -bis * np.sqrt(1.0 / 3.0) - perp * np.sqrt(2.0 / 3.0)
    node_v = jnp.stack([fwdv, bwdv, side], axis=-2) * mask[..., None, None]

    ns8 = jnp.pad(node_s.reshape(N, 7), ((0, 0), (0, 1)))
    nv16 = jnp.pad(node_v.transpose(0, 1, 3, 2).reshape(N, 9),
                   ((0, 0), (0, 7)))
    ca16 = jnp.pad(CA.reshape(N, 3), ((0, 0), (0, 13)))
    ca4 = jnp.pad(CA, ((0, 0), (0, 0), (0, 1)))
    caT = ca4.transpose(0, 2, 1)
    mrow = mask[:, :, None]
    mcol = mask[:, None, :]
    r16 = jnp.pad(_rot_frames(coords).reshape(N, 9), ((0, 0), (0, 7)))

    nbr = _knn(ca4, caT, mrow, mcol)
    boff = jnp.repeat(jnp.arange(B, dtype=jnp.int32) * L, L)[:, None]
    idx2d = (nbr + boff).reshape(E // CHUNK, CHUNK)
    pos = (jnp.arange(N, dtype=jnp.int32) % L)[:, None]
    ang8 = jnp.broadcast_to(
        (nbr - pos).astype(jnp.float32).reshape(E, 1), (E, 8))

    t = _node_embed(ns8, nv16, ca16, params['node_emb'], params['node_ln0'])
    ef = None
    for li, lp in enumerate(params['layers']):
        g = _sc_gather(t, idx2d)
        if li == 0:
            ef = _edge_features(ca16, g, ang8, params['edge_emb'])
        t = _layer(t, g, ef, lp, r16, last=(li == NLAYERS - 1))

    hs = t[:, :NS].reshape(B, L, NS)
    hvr = t[:, NS:PACK].reshape(N, 3, NV).transpose(0, 2, 1).reshape(
        B, L, NV * 3)
    return jnp.concatenate([hs, hvr], -1)


# speedup vs baseline: 1.0180x; 1.0180x over previous
import functools
import numpy as np
import jax
import jax.numpy as jnp
from jax import lax
from jax.experimental import pallas as pl
from jax.experimental.pallas import tpu as pltpu
from jax.experimental.pallas import tpu_sc as plsc

B, L, K = 16, 640, 16
NS, NV = 128, 16
ES, EV = 32, 1
NLAYERS = 3
N = B * L
E = N * K
NB = 128
EB = NB * K
GRID = N // NB
PACK = NS + 3 * NV
TW = 256
EFW = 48
LB = 128
CHUNK = 128
NWORK = 32


def _sc_gather(table, idx2d):
    nchunks, _ = idx2d.shape
    D = table.shape[1]
    cpw = nchunks // NWORK
    mesh = plsc.VectorSubcoreMesh(core_axis_name="c", subcore_axis_name="s")

    @functools.partial(
        pl.kernel, mesh=mesh,
        out_type=jax.ShapeDtypeStruct((nchunks * CHUNK, D), jnp.float32),
        scratch_types=[
            pltpu.VMEM((CHUNK,), jnp.int32),
            pltpu.VMEM((CHUNK, D), jnp.float32),
            pltpu.SemaphoreType.DMA,
        ],
    )
    def k(table_hbm, idx_hbm, out_hbm, idx_v, rows_v, sem):
        wid = lax.axis_index("s") * 2 + lax.axis_index("c")

        def body(i, carry):
            chunk = wid * cpw + i
            pltpu.sync_copy(idx_hbm.at[chunk], idx_v)
            pltpu.async_copy(table_hbm.at[idx_v], rows_v, sem).wait()
            pltpu.sync_copy(rows_v, out_hbm.at[pl.ds(chunk * CHUNK, CHUNK)])
            return carry

        lax.fori_loop(0, cpw, body, 0)

    return k(table, idx2d)


def _full(x):
    nd = x.ndim
    return pl.BlockSpec(x.shape, lambda i, _nd=nd: (0,) * _nd)


def _rep(x):
    d = x.shape[-1]
    return jnp.broadcast_to(x[:, None, :], (NB, K, d)).reshape(NB * K, d)


def _sumk(x):
    d = x.shape[-1]
    return jnp.sum(x.reshape(NB, K, d), axis=1)


def _ln(x, g, b):
    mu = jnp.mean(x, axis=1, keepdims=True)
    xc = x - mu
    var = jnp.mean(xc * xc, axis=1, keepdims=True)
    return g * xc / jnp.sqrt(var + 1e-4) + b


def _lnv(v):
    s2 = v[0] * v[0] + v[1] * v[1] + v[2] * v[2]
    vn = jnp.sqrt(jnp.mean(s2, axis=1, keepdims=True) + 1e-4)
    return [vc / vn for vc in v]


def _knn_body(ca_ref, cat_ref, mr_ref, mc_ref, o_ref):
    ca = ca_ref[0]
    cat = cat_ref[0]
    r = pl.program_id(1)
    d2 = jnp.zeros((LB, L), jnp.float32)
    for c in range(3):
        d = ca[:, c:c + 1] - cat[c:c + 1, :]
        d2 = d2 + d * d
    valid = mr_ref[0] * mc_ref[0]
    rowi = lax.broadcasted_iota(jnp.int32, (LB, L), 0) + r * LB
    coli = lax.broadcasted_iota(jnp.int32, (LB, L), 1)
    cur = jnp.where(valid > 0, d2, 1e10) + jnp.where(coli == rowi, 1e10, 0.0)
    cols = []
    for _ in range(K):
        m = jnp.min(cur, axis=1, keepdims=True)
        idx = jnp.min(jnp.where(cur <= m, coli, L), axis=1, keepdims=True)
        cols.append(idx)
        cur = jnp.where(coli == idx, jnp.float32(3e10), cur)
    o_ref[...] = jnp.concatenate(cols, axis=1)


def _knn(ca4, caT, mrow, mcol):
    rb = L // LB
    return pl.pallas_call(
        _knn_body,
        grid=(B, rb),
        in_specs=[
            pl.BlockSpec((1, LB, 4), lambda b, r: (b, r, 0)),
            pl.BlockSpec((1, 4, L), lambda b, r: (b, 0, 0)),
            pl.BlockSpec((1, LB, 1), lambda b, r: (b, r, 0)),
            pl.BlockSpec((1, 1, L), lambda b, r: (b, 0, 0)),
        ],
        out_specs=pl.BlockSpec((LB, K), lambda b, r, _rb=rb: (b * _rb + r, 0)),
        out_shape=jax.ShapeDtypeStruct((N, K), jnp.int32),
    )(ca4, caT, mrow, mcol)


def _edge_body(cad_ref, g0_ref, ang_ref, wh_ref, wv_ref, wss_ref, wsv_ref,
               bs_ref, wg_ref, bg_ref, o_ref):
    cad = _rep(cad_ref[...])
    off = NS // 2 + _HVW
    cas = g0_ref[...][:, off:off + 3]
    dc = [cas[:, c:c + 1] - cad[:, c:c + 1] for c in range(3)]
    dist = jnp.sqrt(dc[0] * dc[0] + dc[1] * dc[1] + dc[2] * dc[2])
    ev = [d / (dist + 1e-8) for d in dc]
    mu = lax.broadcasted_iota(jnp.int32, (1, 16), 1).astype(jnp.float32) * (
        20.0 / 15.0)
    rbf = jnp.exp(-(((dist - mu) / 1.25) ** 2))
    tvec = lax.broadcasted_iota(jnp.int32, (1, 8), 1).astype(jnp.float32) * 2.0
    freq = jnp.exp(tvec * (-np.log(10000.0) / 16.0))
    ang = ang_ref[...] * freq
    pe = jnp.concatenate([jnp.cos(ang), jnp.sin(ang)], 1)
    es0 = jnp.concatenate([rbf, pe], 1)
    vh = [e * wh_ref[...] for e in ev]
    vn = jnp.sqrt(vh[0] * vh[0] + vh[1] * vh[1] + vh[2] * vh[2] + 1e-8)
    so = jnp.dot(es0, wss_ref[...]) + vn * wsv_ref[...] + bs_ref[...]
    gate = jax.nn.sigmoid(
        jnp.sum(so * wg_ref[...], axis=1, keepdims=True) + bg_ref[...])
    vo = [v * wv_ref[...] * gate for v in vh]
    o_ref[...] = jnp.concatenate(
        [so, vo[0], vo[1], vo[2], jnp.zeros((EB, EFW - 35), jnp.float32)], 1)


def _edge_features(ca16, g0, ang8, ep):
    Ws = ep['Ws']
    wts = [ep['Wh'], ep['Wv'], Ws[:ES], Ws[ES:ES + 1], ep['bs'][None],
           ep['Wg'].T, ep['bg'][None]]
    return pl.pallas_call(
        _edge_body,
        grid=(GRID,),
        in_specs=[
            pl.BlockSpec((NB, 16), lambda i: (i, 0)),
            pl.BlockSpec((EB, GW), lambda i: (i, 0)),
            pl.BlockSpec((EB, 8), lambda i: (i, 0)),
        ] + [_full(w) for w in wts],
        out_specs=pl.BlockSpec((EB, EFW), lambda i: (i, 0)),
        out_shape=jax.ShapeDtypeStruct((E, EFW), jnp.float32),
    )(ca16, g0, ang8, *wts)


GW = 128
_HVW = 3 * NV // 2


def _pack2(x):
    m = x.shape[1] // 2
    xi = lax.bitcast_convert_type(
        x.astype(jnp.bfloat16).astype(jnp.float32), jnp.int32)
    w = jnp.bitwise_or(lax.shift_right_logical(xi[:, :m], 16), xi[:, m:])
    return lax.bitcast_convert_type(w, jnp.float32)


def _unpack2(w):
    wi = lax.bitcast_convert_type(w, jnp.int32)
    lo = lax.shift_left(wi, 16)
    hi = jnp.bitwise_and(wi, jnp.int32(-65536))
    return jnp.concatenate([lax.bitcast_convert_type(lo, jnp.float32),
                            lax.bitcast_convert_type(hi, jnp.float32)], 1)


def _pack_gtable(hs, hv, ca=None):
    parts = [_pack2(hs), _pack2(jnp.concatenate(hv, 1))]
    npad = GW - NS // 2 - _HVW
    if ca is not None:
        parts.append(ca)
        npad -= 3
    parts.append(jnp.zeros((NB, npad), jnp.float32))
    return jnp.concatenate(parts, 1)


def _node_body(ns_ref, nv_ref, ca_ref, wh_ref, wv_ref, wss_ref, wsv_ref,
               bs_ref, wg_ref, bg_ref, g0_ref, b0_ref, o_ref, og_ref):
    ns = ns_ref[...][:, :7]
    v = [nv_ref[...][:, 3 * c:3 * c + 3] for c in range(3)]
    vh = [jnp.dot(vc, wh_ref[...]) for vc in v]
    vn = jnp.sqrt(vh[0] * vh[0] + vh[1] * vh[1] + vh[2] * vh[2] + 1e-8)
    so = jnp.dot(ns, wss_ref[...]) + jnp.dot(vn, wsv_ref[...]) + bs_ref[...]
    vu = [jnp.dot(vhc, wv_ref[...]) for vhc in vh]
    gate = jax.nn.sigmoid(jnp.dot(so, wg_ref[...]) + bg_ref[...])
    vo = [u * gate for u in vu]
    hs = _ln(so, g0_ref[...], b0_ref[...])
    hv = _lnv(vo)
    ca = ca_ref[...][:, :3]
    pad = jnp.zeros((NB, TW - PACK - 3), jnp.float32)
    o_ref[...] = jnp.concatenate([hs] + hv + [ca, pad], 1)
    og_ref[...] = _pack_gtable(hs, hv, ca)


def _node_embed(ns8, nv16, ca16, np_, ln0):
    Ws = np_['Ws']
    wts = [np_['Wh'], np_['Wv'], Ws[:7], Ws[7:], np_['bs'][None],
           np_['Wg'], np_['bg'][None], ln0['g'][None], ln0['b'][None]]
    return pl.pallas_call(
        _node_body,
        grid=(GRID,),
        in_specs=[
            pl.BlockSpec((NB, 8), lambda i: (i, 0)),
            pl.BlockSpec((NB, 16), lambda i: (i, 0)),
            pl.BlockSpec((NB, 16), lambda i: (i, 0)),
        ] + [_full(w) for w in wts],
        out_specs=[pl.BlockSpec((NB, TW), lambda i: (i, 0)),
                   pl.BlockSpec((NB, GW), lambda i: (i, 0))],
        out_shape=[jax.ShapeDtypeStruct((N, TW), jnp.float32),
                   jax.ShapeDtypeStruct((N, GW), jnp.float32)],
    )(ns8, nv16, ca16, *wts)


def _prep_layer(lp):
    w = {}
    m0 = lp['msg'][0]
    Wh, Ws = m0['Wh'], m0['Ws']
    w['m0_Wh_d'] = Wh[:NV]
    w['m0_Wh_s'] = Wh[NV:2 * NV]
    w['m0_wh_e'] = Wh[2 * NV:]
    w['m0_Ws_d'] = Ws[:NS]
    w['m0_Ws_s'] = Ws[NS:2 * NS]
    w['m0_Ws_e'] = Ws[2 * NS:2 * NS + ES]
    w['m0_Ws_v'] = Ws[2 * NS + ES:]
    w['m0_bs'] = m0['bs'][None]
    w['m0_Wv'] = m0['Wv']
    w['m0_Wg'] = m0['Wg']
    w['m0_bg'] = m0['bg'][None]
    for i in (1, 2):
        m, p = lp['msg'][i], f'm{i}'
        w[p + '_Wh'] = m['Wh']
        w[p + '_Ws_s'] = m['Ws'][:NS]
        w[p + '_Ws_v'] = m['Ws'][NS:]
        w[p + '_bs'] = m['bs'][None]
        w[p + '_Wv'] = m['Wv']
        w[p + '_Wg'] = m['Wg']
        w[p + '_bg'] = m['bg'][None]
    for i, (p, si) in enumerate((('f0', NS), ('f1', 2 * NS))):
        m = lp['ff'][i]
        w[p + '_Wh'] = m['Wh']
        w[p + '_Ws_s'] = m['Ws'][:si]
        w[p + '_Ws_v'] = m['Ws'][si:]
        w[p + '_bs'] = m['bs'][None]
        w[p + '_Wv'] = m['Wv']
        w[p + '_Wg'] = m['Wg']
        w[p + '_bg'] = m['bg'][None]
    w['ln1_g'] = lp['ln1']['g'][None]
    w['ln1_b'] = lp['ln1']['b'][None]
    w['ln2_g'] = lp['ln2']['g'][None]
    w['ln2_b'] = lp['ln2']['b'][None]
    for nm in w:
        if ('_Wh' in nm and nm != 'm0_wh_e') or '_Ws_' in nm \
                or nm.endswith('_Wv') or nm.endswith('_Wg'):
            w[nm] = w[nm].astype(jnp.bfloat16)
    return w


def _make_layer_body(wnames, last):
    def body(*refs):
        if last:
            o_ref = refs[-1]
        else:
            o_ref, og_ref = refs[-2], refs[-1]
        tdst = refs[0][...]
        tsrc = refs[1][...]
        ef = refs[2][...]
        W = {nm: refs[3 + i][...] for i, nm in enumerate(wnames)}
        r_ref = refs[3 + len(wnames)] if last else None

        def bdot(a, b):
            return jnp.dot(a.astype(jnp.bfloat16), b,
                           preferred_element_type=jnp.float32)

        def gvp(s, v, p, act):
            vh = [bdot(vc, W[p + '_Wh']) for vc in v]
            vn = jnp.sqrt(vh[0] * vh[0] + vh[1] * vh[1] + vh[2] * vh[2] + 1e-8)
            so = (bdot(s, W[p + '_Ws_s']) + bdot(vn, W[p + '_Ws_v'])
                  + W[p + '_bs'])
            vu = [bdot(vhc, W[p + '_Wv']) for vhc in vh]
            gate = jax.nn.sigmoid(bdot(so, W[p + '_Wg']) + W[p + '_bg'])
            vo = [u * gate for u in vu]
            if act:
                so = jax.nn.relu(so)
            return so, vo

        hs_d = tdst[:, :NS]
        hv_d = [tdst[:, NS + NV * c:NS + NV * (c + 1)] for c in range(3)]
        hs_s = _unpack2(tsrc[:, :NS // 2])
        hv48 = _unpack2(tsrc[:, NS // 2:NS // 2 + _HVW])
        hv_s = [hv48[:, NV * c:NV * (c + 1)] for c in range(3)]
        es = ef[:, :ES]
        ev = [ef[:, ES + c:ES + c + 1] for c in range(3)]

        vh = [_rep(bdot(hv_d[c], W['m0_Wh_d'])) + bdot(hv_s[c], W['m0_Wh_s'])
              + ev[c] * W['m0_wh_e'] for c in range(3)]
        vn = jnp.sqrt(vh[0] * vh[0] + vh[1] * vh[1] + vh[2] * vh[2] + 1e-8)
        so = (_rep(bdot(hs_d, W['m0_Ws_d'])) + bdot(hs_s, W['m0_Ws_s'])
              + bdot(es, W['m0_Ws_e']) + bdot(vn, W['m0_Ws_v'])
              + W['m0_bs'])
        vu = [bdot(vhc, W['m0_Wv']) for vhc in vh]
        gate = jax.nn.sigmoid(bdot(so, W['m0_Wg']) + W['m0_bg'])
        v = [u * gate for u in vu]
        s = jax.nn.relu(so)

        s, v = gvp(s, v, 'm1', True)
        s, v = gvp(s, v, 'm2', False)

        h1 = _ln(hs_d + _sumk(s) * (1.0 / K), W['ln1_g'], W['ln1_b'])
        w1 = _lnv([hv_d[c] + _sumk(v[c]) * (1.0 / K) for c in range(3)])

        fs, fv = gvp(h1, w1, 'f0', True)
        fs, fv = gvp(fs, fv, 'f1', False)
        h2 = _ln(h1 + fs, W['ln2_g'], W['ln2_b'])
        w2 = _lnv([w1[c] + fv[c] for c in range(3)])

        pad = jnp.zeros((NB, TW - PACK), jnp.float32)
        if last:
            R = r_ref[...]
            vr = [w2[0] * R[:, 0 + j:1 + j] + w2[1] * R[:, 3 + j:4 + j]
                  + w2[2] * R[:, 6 + j:7 + j] for j in range(3)]
            o_ref[...] = jnp.concatenate([h2] + vr + [pad], 1)
        else:
            o_ref[...] = jnp.concatenate([h2] + w2 + [pad], 1)
            og_ref[...] = _pack_gtable(h2, w2)

    return body


def _layer(tdst, tsrc, ef, lp, r16, last):
    w = _prep_layer(lp)
    wnames = list(w.keys())
    wvals = [w[nm] for nm in wnames]
    ins = [tdst, tsrc, ef] + wvals
    specs = [
        pl.BlockSpec((NB, TW), lambda i: (i, 0)),
        pl.BlockSpec((EB, GW), lambda i: (i, 0)),
        pl.BlockSpec((EB, EFW), lambda i: (i, 0)),
    ] + [_full(x) for x in wvals]
    if last:
        ins.append(r16)
        specs.append(pl.BlockSpec((NB, 16), lambda i: (i, 0)))
        out_specs = pl.BlockSpec((NB, TW), lambda i: (i, 0))
        out_shape = jax.ShapeDtypeStruct((N, TW), jnp.float32)
    else:
        out_specs = [pl.BlockSpec((NB, TW), lambda i: (i, 0)),
                     pl.BlockSpec((NB, GW), lambda i: (i, 0))]
        out_shape = [jax.ShapeDtypeStruct((N, TW), jnp.float32),
                     jax.ShapeDtypeStruct((N, GW), jnp.float32)]
    return pl.pallas_call(
        _make_layer_body(wnames, last),
        grid=(GRID,),
        in_specs=specs,
        out_specs=out_specs,
        out_shape=out_shape,
    )(*ins)


def _norm(v, eps=1e-8):
    return v / (jnp.linalg.norm(v, axis=-1, keepdims=True) + eps)


def _dihedrals(coords, eps=1e-7):
    X = coords.reshape(coords.shape[0], -1, 3)
    dX = X[:, 1:] - X[:, :-1]
    U = _norm(dX)
    u2, u1, u0 = U[:, :-2], U[:, 1:-1], U[:, 2:]
    n2 = _norm(jnp.cross(u2, u1))
    n1 = _norm(jnp.cross(u1, u0))
    cosD = jnp.clip(jnp.sum(n2 * n1, -1), -1 + eps, 1 - eps)
    D = jnp.sign(jnp.sum(u2 * n1, -1)) * jnp.arccos(cosD)
    D = jnp.pad(D, ((0, 0), (1, 2)))
    D = D.reshape(coords.shape[0], -1, 3)
    return jnp.concatenate([jnp.cos(D), jnp.sin(D)], -1)


def _rot_frames(coords):
    n, ca, c = coords[:, :, 0], coords[:, :, 1], coords[:, :, 2]
    e1 = _norm(c - ca)
    v2 = n - ca
    u2 = v2 - e1 * jnp.sum(e1 * v2, -1, keepdims=True)
    e2 = _norm(u2)
    e3 = jnp.cross(e1, e2)
    return jnp.stack([e1, e2, e3], axis=-2)


def kernel(coords, coord_mask, padding_mask, confidence, params):
    mask = (coord_mask & (~padding_mask)).astype(jnp.float32)
    CA = coords[:, :, 1, :]

    dih = _dihedrals(coords)
    node_s = jnp.concatenate([dih, confidence[..., None]], -1) * mask[..., None]
    fwdv = jnp.pad(_norm(CA[:, 1:] - CA[:, :-1]), ((0, 0), (0, 1), (0, 0)))
    bwdv = jnp.pad(_norm(CA[:, :-1] - CA[:, 1:]), ((0, 0), (1, 0), (0, 0)))
    c_v = _norm(coords[:, :, 2] - CA)
    n_v = _norm(coords[:, :, 0] - CA)
    bis = _norm(c_v + n_v)
    perp = _norm(jnp.cross(c_v, n_v))
    side = -bis * np.sqrt(1.0 / 3.0) - perp * np.sqrt(2.0 / 3.0)
    node_v = jnp.stack([fwdv, bwdv, side], axis=-2) * mask[..., None, None]

    ns8 = jnp.pad(node_s.reshape(N, 7), ((0, 0), (0, 1)))
    nv16 = jnp.pad(node_v.transpose(0, 1, 3, 2).reshape(N, 9),
                   ((0, 0), (0, 7)))
    ca16 = jnp.pad(CA.reshape(N, 3), ((0, 0), (0, 13)))
    ca4 = jnp.pad(CA, ((0, 0), (0, 0), (0, 1)))
    caT = ca4.transpose(0, 2, 1)
    mrow = mask[:, :, None]
    mcol = mask[:, None, :]
    r16 = jnp.pad(_rot_frames(coords).reshape(N, 9), ((0, 0), (0, 7)))

    nbr = _knn(ca4, caT, mrow, mcol)
    boff = jnp.repeat(jnp.arange(B, dtype=jnp.int32) * L, L)[:, None]
    idx2d = (nbr + boff).reshape(E // CHUNK, CHUNK)
    pos = (jnp.arange(N, dtype=jnp.int32) % L)[:, None]
    ang8 = jnp.broadcast_to(
        (nbr - pos).astype(jnp.float32).reshape(E, 1), (E, 8))

    t, gt = _node_embed(ns8, nv16, ca16, params['node_emb'],
                        params['node_ln0'])
    ef = None
    for li, lp in enumerate(params['layers']):
        g = _sc_gather(gt, idx2d)
        if li == 0:
            ef = _edge_features(ca16, g, ang8, params['edge_emb'])
        last = li == NLAYERS - 1
        if last:
            t = _layer(t, g, ef, lp, r16, last=True)
        else:
            t, gt = _layer(t, g, ef, lp, r16, last=False)

    hs = t[:, :NS].reshape(B, L, NS)
    hvr = t[:, NS:PACK].reshape(N, 3, NV).transpose(0, 2, 1).reshape(
        B, L, NV * 3)
    return jnp.concatenate([hs, hvr], -1)
